# Initial kernel scaffold; baseline (speedup 1.0000x reference)
import jax
import jax.numpy as jnp
from jax.experimental import pallas as pl


def kernel(x, pos, edge_index, We1, be1, We2, be2, Wn1, bn1, Wn2, bn2, Wp1, bp1, Wp2, bp2):
    raise NotImplementedError("write your pallas kernel here")


# speedup vs baseline: 2.7587x; 2.7587x over previous
import jax
import jax.numpy as jnp
from jax import lax
from jax.experimental import pallas as pl
from jax.experimental.pallas import tpu as pltpu
from jax.experimental.pallas import tpu_sc as plsc

N = 10000
E = 320000
D = 128
P = 3
L = 3

NPAD = 10240
NC, NS = 2, 16
NW = NC * NS
CH = 128
SLAB = NPAD // NS

EPW = ((E + NW * CH - 1) // (NW * CH)) * CH
EP = EPW * NW
GCH = EPW // CH
EPT = EP // NS
SCH = EPT // CH

EB = 1024
NB = 1024


def _mm(a, b):
    return lax.dot_general(a, b, (((1,), (0,)), ((), ())))


def _sig(x):
    return 1.0 / (1.0 + jnp.exp(-x))


def _sc_mesh():
    return plsc.VectorSubcoreMesh(
        core_axis_name="c", subcore_axis_name="s", num_cores=NC, num_subcores=NS)


def _gather_body(t_hbm, pos4_hbm, dst_hbm, src_hbm, gi_hbm, gj_hbm, relw_hbm,
                 pos_v, idxd, idxs, rows_i, rows_j, r0, r1, r2, r3, sem):
    cid = lax.axis_index("c")
    sid = lax.axis_index("s")
    wid = sid * NC + cid
    base = wid * EPW

    pltpu.sync_copy(pos4_hbm, pos_v)
    rbufs = [r0, r1, r2, r3]

    def chunk(j, _):
        off = base + j * CH
        pltpu.sync_copy(dst_hbm.at[pl.ds(off, CH)], idxd)
        pltpu.sync_copy(src_hbm.at[pl.ds(off, CH)], idxs)
        pltpu.async_copy(t_hbm.at[idxd], rows_i, sem).wait()
        pltpu.async_copy(t_hbm.at[idxs], rows_j, sem).wait()
        for g in range(CH // 16):
            sl = pl.ds(g * 16, 16)
            d16 = idxd[sl]
            s16 = idxs[sl]
            d2 = jnp.zeros((16,), jnp.float32)
            for c in range(P):
                pc = plsc.load_gather(pos_v, [d16 + c * NPAD])
                qc = plsc.load_gather(pos_v, [s16 + c * NPAD])
                rc = pc - qc
                rbufs[c][sl] = rc
                d2 = d2 + rc * rc
            r3[sl] = d2
        pltpu.sync_copy(rows_i, gi_hbm.at[pl.ds(off, CH)])
        pltpu.sync_copy(rows_j, gj_hbm.at[pl.ds(off, CH)])
        for c in range(4):
            pltpu.sync_copy(rbufs[c], relw_hbm.at[c, pl.ds(off, CH)])
        return _

    lax.fori_loop(0, GCH, chunk, 0)


def _sc_gather(t, pos4_flat, dst_p, src_p):
    fn = pl.kernel(
        _gather_body,
        out_type=(jax.ShapeDtypeStruct((EP, D), jnp.float32),
                  jax.ShapeDtypeStruct((EP, D), jnp.float32),
                  jax.ShapeDtypeStruct((4, EP), jnp.float32)),
        mesh=_sc_mesh(),
        compiler_params=pltpu.CompilerParams(needs_layout_passes=False),
        scratch_types=[
            pltpu.VMEM((4 * NPAD,), jnp.float32),
            pltpu.VMEM((CH,), jnp.int32),
            pltpu.VMEM((CH,), jnp.int32),
            pltpu.VMEM((CH, D), jnp.float32),
            pltpu.VMEM((CH, D), jnp.float32),
            pltpu.VMEM((CH,), jnp.float32),
            pltpu.VMEM((CH,), jnp.float32),
            pltpu.VMEM((CH,), jnp.float32),
            pltpu.VMEM((CH,), jnp.float32),
            pltpu.SemaphoreType.DMA,
        ],
    )
    return fn(t, pos4_flat, dst_p, src_p)


def _scatter_body(m_hbm, tc_hbm, dst_hbm, zero_hbm,
                  outm_hbm, outtc_hbm, idx_v, rows_v, acc, sem):
    cid = lax.axis_index("c")
    sid = lax.axis_index("s")

    pltpu.sync_copy(zero_hbm.at[pl.ds(sid * SLAB, SLAB)],
                    acc.at[pl.ds(sid * SLAB, SLAB)])
    plsc.subcore_barrier()

    def chunk(j, _):
        off = sid * EPT + j * CH
        pltpu.sync_copy(dst_hbm.at[pl.ds(off, CH)], idx_v.at[0])

        @pl.when(cid == 0)
        def _rd_m():
            pltpu.sync_copy(m_hbm.at[pl.ds(off, CH)], rows_v)

        @pl.when(cid == 1)
        def _rd_tc():
            pltpu.sync_copy(tc_hbm.at[pl.ds(off, CH)], rows_v)

        pltpu.sync_copy(rows_v, acc.at[idx_v.at[0]], add=True)
        return _

    lax.fori_loop(0, SCH, chunk, 0)
    plsc.subcore_barrier()

    @pl.when(cid == 0)
    def _out_m():
        pltpu.sync_copy(acc.at[pl.ds(sid * SLAB, SLAB)],
                        outm_hbm.at[pl.ds(sid * SLAB, SLAB)])

    @pl.when(cid == 1)
    def _out_tc():
        pltpu.sync_copy(acc.at[pl.ds(sid * SLAB, SLAB)],
                        outtc_hbm.at[pl.ds(sid * SLAB, SLAB)])


def _sc_scatter(m, tcrows, dst_p, zeros_np):
    fn = pl.kernel(
        _scatter_body,
        out_type=(jax.ShapeDtypeStruct((NPAD, D), jnp.float32),
                  jax.ShapeDtypeStruct((NPAD, D), jnp.float32)),
        mesh=_sc_mesh(),
        scratch_types=[
            pltpu.VMEM((1, CH), jnp.int32),
            pltpu.VMEM((CH, D), jnp.float32),
            pltpu.VMEM_SHARED((NPAD, D), jnp.float32),
            pltpu.SemaphoreType.DMA,
        ],
    )
    return fn(m, tcrows, dst_p, zeros_np)


def _edge_body(gi, gj, relw, w1i, w1j, w1d, b1, w2, b2, wp1, bp1, wp2, bp2,
               m_out, tc_out):
    xi = gi[...]
    xj = gj[...]
    rw = relw[...]
    e3 = jnp.where(lax.broadcasted_iota(jnp.int32, (4, 1), 0) == 3, 1.0, 0.0)
    d2 = lax.dot_general(rw, e3, (((0,), (0,)), ((), ())))
    h = _mm(xi, w1i[...]) + _mm(xj, w1j[...]) + d2 * w1d[0] + b1[0]
    h = h * _sig(h)
    m = _mm(h, w2[...]) + b2[0]
    m = m * _sig(m)
    hp = _mm(m, wp1[...]) + bp1[0]
    hp = hp * _sig(hp)
    wv = _mm(hp, wp2[...])[:, :1] + bp2[0, 0]
    eid = pl.program_id(0) * EB + lax.broadcasted_iota(jnp.int32, (EB, 1), 0)
    mask = (eid < E).astype(jnp.float32)
    rio = lax.broadcasted_iota(jnp.int32, (4, D), 0)
    cio = lax.broadcasted_iota(jnp.int32, (4, D), 1)
    sel = jnp.where((rio == cio) & (rio < P), 1.0, 0.0)
    rel128 = lax.dot_general(rw, sel, (((0,), (0,)), ((), ())))
    cnt_row = jnp.where(
        lax.broadcasted_iota(jnp.int32, (1, D), 1) == P, 1.0, 0.0)
    m_out[...] = m * mask
    tc_out[...] = rel128 * (wv * mask) + mask * cnt_row


def _tc_edge(gi, gj, relw, w1i, w1j, w1d, b1, w2, b2, wp1, bp1, wp2, bp2):
    full = lambda shape: pl.BlockSpec(shape, lambda i: (0,) * len(shape))
    return pl.pallas_call(
        _edge_body,
        grid=(EP // EB,),
        in_specs=[
            pl.BlockSpec((EB, D), lambda i: (i, 0)),
            pl.BlockSpec((EB, D), lambda i: (i, 0)),
            pl.BlockSpec((4, EB), lambda i: (0, i)),
            full((D, D)), full((D, D)), full((1, D)), full((1, D)),
            full((D, D)), full((1, D)), full((D, D)), full((1, D)),
            full((D, D)), full((1, 1)),
        ],
        out_specs=[pl.BlockSpec((EB, D), lambda i: (i, 0)),
                   pl.BlockSpec((EB, D), lambda i: (i, 0))],
        out_shape=[jax.ShapeDtypeStruct((EP, D), jnp.float32),
                   jax.ShapeDtypeStruct((EP, D), jnp.float32)],
    )(gi, gj, relw, w1i, w1j, w1d, b1, w2, b2, wp1, bp1, wp2, bp2)


def _node_body(t, om, otc, post, wn1x, wn1a, bn1, wn2, bn2, tout, post_out):
    x = t[...]
    agg = om[...]
    h = _mm(x, wn1x[...]) + _mm(agg, wn1a[...]) + bn1[0]
    h = h * _sig(h)
    tout[...] = _mm(h, wn2[...]) + bn2[0]

    sel = jnp.where(
        lax.broadcasted_iota(jnp.int32, (8, D), 0)
        == lax.broadcasted_iota(jnp.int32, (8, D), 1), 1.0, 0.0)
    nt = lax.dot_general(sel, otc[...], (((1,), (1,)), ((), ())))
    cnt = jnp.maximum(nt[P:P + 1, :], 1.0)
    i = pl.program_id(0)
    sl = pl.ds(i * NB, NB)
    post_out[:, sl] = post[:, sl] + nt * (1.0 / cnt)


def _tc_node(t, out_m, out_tc, pos_t, wn1x, wn1a, bn1, wn2, bn2):
    full = lambda shape: pl.BlockSpec(shape, lambda i: (0,) * len(shape))
    return pl.pallas_call(
        _node_body,
        grid=(NPAD // NB,),
        in_specs=[
            pl.BlockSpec((NB, D), lambda i: (i, 0)),
            pl.BlockSpec((NB, D), lambda i: (i, 0)),
            pl.BlockSpec((NB, D), lambda i: (i, 0)),
            full((8, NPAD)),
            full((D, D)), full((D, D)), full((1, D)),
            full((D, D)), full((1, D)),
        ],
        out_specs=[pl.BlockSpec((NB, D), lambda i: (i, 0)),
                   full((8, NPAD))],
        out_shape=[jax.ShapeDtypeStruct((NPAD, D), jnp.float32),
                   jax.ShapeDtypeStruct((8, NPAD), jnp.float32)],
    )(t, out_m, out_tc, pos_t, wn1x, wn1a, bn1, wn2, bn2)


def kernel(x, pos, edge_index, We1, be1, We2, be2, Wn1, bn1, Wn2, bn2,
           Wp1, bp1, Wp2, bp2):
    src = edge_index[0].astype(jnp.int32)
    dst = edge_index[1].astype(jnp.int32)
    pad = EP - E
    dst_p = jnp.concatenate([dst, jnp.zeros((pad,), jnp.int32)])
    src_p = jnp.concatenate([src, jnp.zeros((pad,), jnp.int32)])
    zeros_np = jnp.zeros((NPAD, D), jnp.float32)

    t = jnp.concatenate([x, jnp.zeros((NPAD - N, D), jnp.float32)])
    pos_t = jnp.concatenate(
        [pos.T, jnp.zeros((8 - P, N), jnp.float32)])
    pos_t = jnp.concatenate(
        [pos_t, jnp.zeros((8, NPAD - N), jnp.float32)], axis=1)

    for l in range(L):
        w1i = We1[l, :D]
        w1j = We1[l, D:2 * D]
        w1d = We1[l, 2 * D:2 * D + 1]
        wp2 = jnp.pad(Wp2[l], ((0, 0), (0, D - 1)))
        pos4_flat = pos_t[:4].reshape(4 * NPAD)
        gi, gj, relw = _sc_gather(t, pos4_flat, dst_p, src_p)
        m, tcrows = _tc_edge(gi, gj, relw, w1i, w1j, w1d, be1[l][None],
                             We2[l], be2[l][None], Wp1[l], bp1[l][None],
                             wp2, bp2[l][None])
        out_m, out_tc = _sc_scatter(m, tcrows, dst_p, zeros_np)
        t, pos_t = _tc_node(t, out_m, out_tc, pos_t, Wn1[l, :D], Wn1[l, D:],
                            bn1[l][None], Wn2[l], bn2[l][None])

    return (t[:N], pos_t[:P, :N].T)
